# baseline (speedup 1.0000x reference)
import jax
import jax.numpy as jnp
from jax import lax
from jax.experimental import pallas as pl

K = 20
ROWS = 1024
COLS = 100000
BLOCK_ROWS = 8


def _topk_body(x_ref, out_ref):
    x = x_ref[...]
    col = lax.broadcasted_iota(jnp.int32, (BLOCK_ROWS, COLS), 1)
    kcol = lax.broadcasted_iota(jnp.int32, (BLOCK_ROWS, K), 1)

    def step(k, carry):
        xc, out = carry
        m = jnp.min(xc, axis=1, keepdims=True)
        idx = jnp.min(jnp.where(xc == m, col, COLS), axis=1, keepdims=True)
        out = jnp.where(kcol == k, idx, out)
        xc = jnp.where((xc == m) & (col == idx), jnp.inf, xc)
        return xc, out

    out0 = jnp.zeros((BLOCK_ROWS, K), jnp.int32)
    _, out = lax.fori_loop(0, K, step, (x, out0))
    out_ref[...] = out


def kernel(inputs):
    return pl.pallas_call(
        _topk_body,
        grid=(ROWS // BLOCK_ROWS,),
        in_specs=[pl.BlockSpec((BLOCK_ROWS, COLS), lambda i: (i, 0))],
        out_specs=pl.BlockSpec((BLOCK_ROWS, K), lambda i: (i, 0)),
        out_shape=jax.ShapeDtypeStruct((ROWS, K), jnp.int32),
    )(inputs)
